# Initial kernel scaffold; baseline (speedup 1.0000x reference)
import jax
import jax.numpy as jnp
from jax.experimental import pallas as pl


def kernel(x, edge_index, batch, W1, a_src1, a_dst1, b1, W2, a_src2, a_dst2, b2, lin_W, lin_b):
    raise NotImplementedError("write your pallas kernel here")


# speedup vs baseline: 1.0899x; 1.0899x over previous
import functools

import jax
import jax.numpy as jnp
from jax import lax
from jax.experimental import pallas as pl
from jax.experimental.pallas import tpu as pltpu

N_NODES = 10000
N_EDGES = 320000
D_FEAT = 128
HIDDEN = 64
HEADS = 8
N_GRAPHS = 64
N_CLASSES = 10

ROW_BLK = 1000


def _mm1_body(x_ref, w_ref, as_ref, ad_ref, h_ref, a1_ref, a2_ref):
    h = jnp.dot(x_ref[...], w_ref[...], preferred_element_type=jnp.float32)
    h_ref[...] = h
    a1_ref[...] = jnp.dot(h, as_ref[...], preferred_element_type=jnp.float32)
    a2_ref[...] = jnp.dot(h, ad_ref[...], preferred_element_type=jnp.float32)


def _mm1(x, W1, asrcM, adstM):
    n = x.shape[0]
    grid = n // ROW_BLK
    return pl.pallas_call(
        _mm1_body,
        grid=(grid,),
        in_specs=[
            pl.BlockSpec((ROW_BLK, D_FEAT), lambda i: (i, 0)),
            pl.BlockSpec((D_FEAT, HEADS * HIDDEN), lambda i: (0, 0)),
            pl.BlockSpec((HEADS * HIDDEN, 128), lambda i: (0, 0)),
            pl.BlockSpec((HEADS * HIDDEN, 128), lambda i: (0, 0)),
        ],
        out_specs=[
            pl.BlockSpec((ROW_BLK, HEADS * HIDDEN), lambda i: (i, 0)),
            pl.BlockSpec((ROW_BLK, 128), lambda i: (i, 0)),
            pl.BlockSpec((ROW_BLK, 128), lambda i: (i, 0)),
        ],
        out_shape=[
            jax.ShapeDtypeStruct((n, HEADS * HIDDEN), jnp.float32),
            jax.ShapeDtypeStruct((n, 128), jnp.float32),
            jax.ShapeDtypeStruct((n, 128), jnp.float32),
        ],
    )(x, W1, asrcM, adstM)


def _gat_rest(h, edge_index, heads, out_ch, alpha_src, alpha_dst, bias, concat):
    N = h.shape[0]
    src = edge_index[0]
    dst = edge_index[1]
    hr = h.reshape(N, heads, out_ch)
    a_s = (hr * alpha_src[None, :, :]).sum(-1)
    a_d = (hr * alpha_dst[None, :, :]).sum(-1)
    e = a_s[src] + a_d[dst]
    e = jax.nn.leaky_relu(e, negative_slope=0.2)
    ex = jnp.exp(e)
    denom = jax.ops.segment_sum(ex, dst, num_segments=N)
    alpha = ex / (denom[dst] + 1e-16)
    msg = hr[src] * alpha[:, :, None]
    out = jax.ops.segment_sum(msg, dst, num_segments=N)
    if concat:
        out = out.reshape(N, heads * out_ch)
    else:
        out = out.mean(axis=1)
    return out + bias, alpha


def kernel(x, edge_index, batch, W1, a_src1, a_dst1, b1, W2, a_src2, a_dst2, b2, lin_W, lin_b):
    eye = jnp.eye(HEADS, dtype=jnp.float32)
    asrcM = (eye[:, None, :] * a_src1[:, :, None]).reshape(HEADS * HIDDEN, HEADS)
    adstM = (eye[:, None, :] * a_dst1[:, :, None]).reshape(HEADS * HIDDEN, HEADS)
    asrcM = jnp.pad(asrcM, ((0, 0), (0, 128 - HEADS)))
    adstM = jnp.pad(adstM, ((0, 0), (0, 128 - HEADS)))

    h1pre, _, _ = _mm1(x, W1, asrcM, adstM)
    h1, _ = _gat_rest(h1pre, edge_index, HEADS, HIDDEN, a_src1, a_dst1, b1, True)
    h1 = jax.nn.elu(h1)
    h2, alpha = _gat_rest(h1 @ W2, edge_index, 1, HIDDEN, a_src2, a_dst2, b2, False)
    h2 = jax.nn.elu(h2)
    counts = jax.ops.segment_sum(jnp.ones((h2.shape[0],), jnp.float32), batch, num_segments=N_GRAPHS)
    pooled = jax.ops.segment_sum(h2, batch, num_segments=N_GRAPHS) / jnp.maximum(counts, 1.0)[:, None]
    logits = pooled @ lin_W + lin_b
    logp = jax.nn.log_softmax(logits, axis=1)
    return logp, (edge_index, alpha)
